# baseline (speedup 1.0000x reference)
import dataclasses
import functools

import jax
import jax.numpy as jnp
from jax import lax
from jax.experimental import pallas as pl
from jax.experimental.pallas import tpu as pltpu
from jax.experimental.pallas import tpu_sc as plsc

B = 16384
D = 32
L = 16
NC = 2
NS = 16
NW = NC * NS
BPW = B // NW
V = 1000001


def _gmf_body(uid_hbm, iid_hbm, ubias_hbm, ibias_hbm, gb_hbm, uemb1_hbm,
              iemb1_hbm, wb_hbm, b_hbm, out_hbm,
              uid_v, iid_v, uidx_v, iidx_v, ut_v, it_v, ub_v, ib_v, wb_v,
              gb_v, b_v, out_v, sem_u, sem_i, sem_ub, sem_ib):
  wid = lax.axis_index("s") * NC + lax.axis_index("c")
  base = wid * BPW

  pltpu.sync_copy(wb_hbm, wb_v)
  pltpu.sync_copy(gb_hbm, gb_v)
  pltpu.sync_copy(b_hbm, b_v)

  pltpu.sync_copy(uid_hbm.at[pl.ds(base, BPW)], uid_v)
  pltpu.sync_copy(iid_hbm.at[pl.ds(base, BPW)], iid_v)

  cub = pltpu.async_copy(ubias_hbm.at[uid_v], ub_v, sem_ub)
  cib = pltpu.async_copy(ibias_hbm.at[iid_v], ib_v, sem_ib)

  for d in range(D):
    off = jnp.full((L,), d * V, dtype=jnp.int32)

    @pl.loop(0, BPW, step=L)
    def _(j0, off=off, d=d):
      sl = pl.ds(j0, L)
      dst = pl.ds(d * BPW + j0, L)
      uidx_v[dst] = uid_v[sl] + off
      iidx_v[dst] = iid_v[sl] + off

  cu = pltpu.async_copy(uemb1_hbm.at[uidx_v], ut_v, sem_u)
  ci = pltpu.async_copy(iemb1_hbm.at[iidx_v], it_v, sem_i)
  cu.wait()
  ci.wait()
  cub.wait()
  cib.wait()

  const = gb_v[...] + b_v[...]

  @pl.loop(0, BPW, step=L)
  def _(k0):
    sl = pl.ds(k0, L)
    acc = ub_v[sl] + ib_v[sl] + const
    for d in range(D):
      acc = acc + (ut_v[pl.ds(d * BPW + k0, L)] * it_v[pl.ds(d * BPW + k0, L)]
                   * wb_v[d, pl.ds(0, L)])
    out_v[sl] = acc

  pltpu.sync_copy(out_v, out_hbm.at[pl.ds(base, BPW)])


@jax.jit
def kernel(user_id, item_id, user_bias, item_bias, global_bias, user_emb,
           item_emb, W, b):
  mesh = plsc.VectorSubcoreMesh(core_axis_name="c", subcore_axis_name="s",
                                num_cores=NC, num_subcores=NS)
  cp = pltpu.CompilerParams(needs_layout_passes=False,
                            use_tc_tiling_on_sc=False)
  run = pl.kernel(
      _gmf_body,
      out_type=jax.ShapeDtypeStruct((B,), jnp.float32),
      mesh=mesh,
      compiler_params=cp,
      scratch_types=[
          pltpu.VMEM((BPW,), jnp.int32),
          pltpu.VMEM((BPW,), jnp.int32),
          pltpu.VMEM((D * BPW,), jnp.int32),
          pltpu.VMEM((D * BPW,), jnp.int32),
          pltpu.VMEM((D * BPW,), jnp.float32),
          pltpu.VMEM((D * BPW,), jnp.float32),
          pltpu.VMEM((BPW,), jnp.float32),
          pltpu.VMEM((BPW,), jnp.float32),
          pltpu.VMEM((D, L), jnp.float32),
          pltpu.VMEM((L,), jnp.float32),
          pltpu.VMEM((L,), jnp.float32),
          pltpu.VMEM((BPW,), jnp.float32),
          pltpu.SemaphoreType.DMA,
          pltpu.SemaphoreType.DMA,
          pltpu.SemaphoreType.DMA,
          pltpu.SemaphoreType.DMA,
      ],
  )
  gb16 = jnp.broadcast_to(global_bias.reshape(()), (L,))
  b16 = jnp.broadcast_to(b.reshape(()), (L,))
  wb = jnp.broadcast_to(W.reshape(D, 1), (D, L))
  u1 = user_emb.T.reshape(-1)
  i1 = item_emb.T.reshape(-1)
  return run(user_id.astype(jnp.int32), item_id.astype(jnp.int32),
             user_bias, item_bias, gb16, u1, i1, wb, b16)


# speedup vs baseline: 3.7072x; 3.7072x over previous
import dataclasses
import functools

import jax
import jax.numpy as jnp
from jax import lax
from jax.experimental import pallas as pl
from jax.experimental.pallas import tpu as pltpu
from jax.experimental.pallas import tpu_sc as plsc

B = 16384
D = 32
L = 16
NC = 2
NS = 16
NW = NC * NS
BPW = B // NW


def _gmf_body(uid_hbm, iid_hbm, ubias_hbm, ibias_hbm, gb_hbm, uemb_hbm,
              iemb_hbm, w_hbm, b_hbm, idxt_hbm, out_hbm,
              uid_v, iid_v, urows_v, irows_v, ub_v, ib_v, w_v, gb_v, b_v,
              idxt_v, tbuf_v, out_v, sem_u, sem_i, sem_ub, sem_ib):
  wid = lax.axis_index("s") * NC + lax.axis_index("c")
  base = wid * BPW

  pltpu.sync_copy(w_hbm, w_v)
  pltpu.sync_copy(gb_hbm, gb_v)
  pltpu.sync_copy(b_hbm, b_v)
  pltpu.sync_copy(idxt_hbm, idxt_v)

  pltpu.sync_copy(uid_hbm.at[pl.ds(base, BPW)], uid_v)
  pltpu.sync_copy(iid_hbm.at[pl.ds(base, BPW)], iid_v)

  cu = pltpu.async_copy(uemb_hbm.at[uid_v], urows_v, sem_u)
  ci = pltpu.async_copy(iemb_hbm.at[iid_v], irows_v, sem_i)
  cub = pltpu.async_copy(ubias_hbm.at[uid_v], ub_v, sem_ub)
  cib = pltpu.async_copy(ibias_hbm.at[iid_v], ib_v, sem_ib)
  cu.wait()
  ci.wait()
  cub.wait()
  cib.wait()

  w0 = w_v[pl.ds(0, L)]
  w1 = w_v[pl.ds(L, L)]
  const = gb_v[...] + b_v[...]

  @pl.loop(0, BPW, step=L)
  def _(k0):
    sl = pl.ds(k0, L)
    for r in range(L):
      k = k0 + r
      u0, u1 = plsc.unpack(urows_v[k, pl.ds(0, D)],
                           format=plsc.PackFormat.INTERLEAVED)
      i0, i1 = plsc.unpack(irows_v[k, pl.ds(0, D)],
                           format=plsc.PackFormat.INTERLEAVED)
      tbuf_v[pl.ds(r * L, L)] = u0 * i0 * w0 + u1 * i1 * w1
    acc = ub_v[sl] + ib_v[sl] + const
    for d in range(L):
      acc = acc + plsc.load_gather(tbuf_v, [idxt_v[d, pl.ds(0, L)]])
    out_v[sl] = acc

  pltpu.sync_copy(out_v, out_hbm.at[pl.ds(base, BPW)])


@jax.jit
def kernel(user_id, item_id, user_bias, item_bias, global_bias, user_emb,
           item_emb, W, b):
  mesh = plsc.VectorSubcoreMesh(core_axis_name="c", subcore_axis_name="s",
                                num_cores=NC, num_subcores=NS)
  cp = pltpu.CompilerParams(needs_layout_passes=False,
                            use_tc_tiling_on_sc=False)
  run = pl.kernel(
      _gmf_body,
      out_type=jax.ShapeDtypeStruct((B,), jnp.float32),
      mesh=mesh,
      compiler_params=cp,
      scratch_types=[
          pltpu.VMEM((BPW,), jnp.int32),
          pltpu.VMEM((BPW,), jnp.int32),
          pltpu.VMEM((BPW, D), jnp.bfloat16),
          pltpu.VMEM((BPW, D), jnp.bfloat16),
          pltpu.VMEM((BPW,), jnp.float32),
          pltpu.VMEM((BPW,), jnp.float32),
          pltpu.VMEM((D,), jnp.float32),
          pltpu.VMEM((L,), jnp.float32),
          pltpu.VMEM((L,), jnp.float32),
          pltpu.VMEM((L, L), jnp.int32),
          pltpu.VMEM((L * L,), jnp.float32),
          pltpu.VMEM((BPW,), jnp.float32),
          pltpu.SemaphoreType.DMA,
          pltpu.SemaphoreType.DMA,
          pltpu.SemaphoreType.DMA,
          pltpu.SemaphoreType.DMA,
      ],
  )
  gb16 = jnp.broadcast_to(global_bias.reshape(()), (L,))
  b16 = jnp.broadcast_to(b.reshape(()), (L,))
  idxt = (jnp.arange(L, dtype=jnp.int32)[:, None]
          + L * jnp.arange(L, dtype=jnp.int32)[None, :])
  w_flat = W.reshape(D)
  w_de = jnp.concatenate([w_flat[0::2], w_flat[1::2]])
  return run(user_id.astype(jnp.int32), item_id.astype(jnp.int32),
             user_bias, item_bias, gb16,
             user_emb.astype(jnp.bfloat16), item_emb.astype(jnp.bfloat16),
             w_de, b16, idxt)


# speedup vs baseline: 5.7014x; 1.5379x over previous
import dataclasses
import functools

import jax
import jax.numpy as jnp
from jax import lax
from jax.experimental import pallas as pl
from jax.experimental.pallas import tpu as pltpu
from jax.experimental.pallas import tpu_sc as plsc

B = 16384
D = 32
L = 16
NC = 2
NS = 16
NW = NC * NS
BPW = B // NW


def _gmf_body(uid_hbm, iid_hbm, ubias_hbm, ibias_hbm, gb_hbm, uemb_hbm,
              iemb_hbm, w_hbm, b_hbm, idxt_hbm, out_hbm,
              uid_v, iid_v, urows_v, irows_v, ub_v, ib_v, w_v, gb_v, b_v,
              idxt_v, tbuf_v, out_v, sem_u, sem_i, sem_ub, sem_ib):
  wid = lax.axis_index("s") * NC + lax.axis_index("c")
  base = wid * BPW

  pltpu.sync_copy(w_hbm, w_v)
  pltpu.sync_copy(gb_hbm, gb_v)
  pltpu.sync_copy(b_hbm, b_v)
  pltpu.sync_copy(idxt_hbm, idxt_v)

  pltpu.sync_copy(uid_hbm.at[pl.ds(base, BPW)], uid_v)
  pltpu.sync_copy(iid_hbm.at[pl.ds(base, BPW)], iid_v)

  cu = pltpu.async_copy(uemb_hbm.at[uid_v], urows_v, sem_u)
  ci = pltpu.async_copy(iemb_hbm.at[iid_v], irows_v, sem_i)
  cub = pltpu.async_copy(ubias_hbm.at[uid_v], ub_v, sem_ub)
  cib = pltpu.async_copy(ibias_hbm.at[iid_v], ib_v, sem_ib)
  cu.wait()
  ci.wait()
  cub.wait()
  cib.wait()

  w0 = w_v[pl.ds(0, L)]
  w1 = w_v[pl.ds(L, L)]
  const = gb_v[...] + b_v[...]

  @pl.loop(0, BPW, step=L)
  def _(k0):
    sl = pl.ds(k0, L)
    for r in range(L):
      k = k0 + r
      u0 = urows_v[k, pl.ds(0, L)]
      u1 = urows_v[k, pl.ds(L, L)]
      i0 = irows_v[k, pl.ds(0, L)]
      i1 = irows_v[k, pl.ds(L, L)]
      tbuf_v[pl.ds(r * L, L)] = u0 * i0 * w0 + u1 * i1 * w1
    acc = ub_v[sl] + ib_v[sl] + const
    for d in range(L):
      acc = acc + plsc.load_gather(tbuf_v, [idxt_v[d, pl.ds(0, L)]])
    out_v[sl] = acc

  pltpu.sync_copy(out_v, out_hbm.at[pl.ds(base, BPW)])


@jax.jit
def kernel(user_id, item_id, user_bias, item_bias, global_bias, user_emb,
           item_emb, W, b):
  mesh = plsc.VectorSubcoreMesh(core_axis_name="c", subcore_axis_name="s",
                                num_cores=NC, num_subcores=NS)
  cp = pltpu.CompilerParams(needs_layout_passes=False,
                            use_tc_tiling_on_sc=False)
  run = pl.kernel(
      _gmf_body,
      out_type=jax.ShapeDtypeStruct((B,), jnp.float32),
      mesh=mesh,
      compiler_params=cp,
      scratch_types=[
          pltpu.VMEM((BPW,), jnp.int32),
          pltpu.VMEM((BPW,), jnp.int32),
          pltpu.VMEM((BPW, D), jnp.float32),
          pltpu.VMEM((BPW, D), jnp.float32),
          pltpu.VMEM((BPW,), jnp.float32),
          pltpu.VMEM((BPW,), jnp.float32),
          pltpu.VMEM((D,), jnp.float32),
          pltpu.VMEM((L,), jnp.float32),
          pltpu.VMEM((L,), jnp.float32),
          pltpu.VMEM((L, L), jnp.int32),
          pltpu.VMEM((L * L,), jnp.float32),
          pltpu.VMEM((BPW,), jnp.float32),
          pltpu.SemaphoreType.DMA,
          pltpu.SemaphoreType.DMA,
          pltpu.SemaphoreType.DMA,
          pltpu.SemaphoreType.DMA,
      ],
  )
  gb16 = jnp.broadcast_to(global_bias.reshape(()), (L,))
  b16 = jnp.broadcast_to(b.reshape(()), (L,))
  idxt = (jnp.arange(L, dtype=jnp.int32)[:, None]
          + L * jnp.arange(L, dtype=jnp.int32)[None, :])
  return run(user_id.astype(jnp.int32), item_id.astype(jnp.int32),
             user_bias, item_bias, gb16, user_emb, item_emb,
             W.reshape(D), b16, idxt)
